# baseline (speedup 1.0000x reference)
import functools

import jax
import jax.numpy as jnp
from jax import lax
from jax.experimental import pallas as pl
from jax.experimental.pallas import tpu as pltpu
from jax.experimental.pallas import tpu_sc as plsc

VOCAB = 30000
HIDDEN = 768
BATCH = 4
SEQ = 4096
TOTAL = BATCH * SEQ

_info = plsc.get_sparse_core_info()
NC, NS = _info.num_cores, _info.num_subcores
NW = NC * NS
ROWS_PER_W = TOTAL // NW
W_PER_B = SEQ // ROWS_PER_W
CHUNK = 32
NCHUNK = ROWS_PER_W // CHUNK
NBUF = 4
NGROUP = NCHUNK // NBUF

_mesh = plsc.VectorSubcoreMesh(core_axis_name="c", subcore_axis_name="s")


@functools.partial(
    pl.kernel,
    mesh=_mesh,
    out_type=jax.ShapeDtypeStruct((BATCH, SEQ, HIDDEN), jnp.float32),
    scratch_types=[
        pltpu.VMEM((ROWS_PER_W,), jnp.int32),
        pltpu.VMEM((NBUF, CHUNK, HIDDEN), jnp.float32),
        pltpu.SemaphoreType.DMA((NBUF,)),
        pltpu.SemaphoreType.DMA((NBUF,)),
    ],
)
def _gather(ids_hbm, table_hbm, out_hbm, idx_v, rows_v, gsem, psem):
    wid = lax.axis_index("s") * NC + lax.axis_index("c")
    b = wid // W_PER_B
    s_off = (wid % W_PER_B) * ROWS_PER_W
    pltpu.sync_copy(ids_hbm.at[b, pl.ds(s_off, ROWS_PER_W)], idx_v)

    def g_copy(c, buf):
        return pltpu.make_async_copy(
            table_hbm.at[idx_v.at[pl.ds(c * CHUNK, CHUNK)]], rows_v.at[buf],
            gsem.at[buf])

    def p_copy(c, buf):
        return pltpu.make_async_copy(
            rows_v.at[buf], out_hbm.at[b, pl.ds(s_off + c * CHUNK, CHUNK)],
            psem.at[buf])

    for buf in range(NBUF):
        g_copy(buf, buf).start()

    def group_body(g, carry):
        for buf in range(NBUF):
            c = g * NBUF + buf
            g_copy(c, buf).wait()
            p_copy(c, buf).start()
        for buf in range(NBUF):
            c = g * NBUF + buf
            p_copy(c, buf).wait()
            g_copy(c + NBUF, buf).start()
        return carry

    lax.fori_loop(0, NGROUP - 1, group_body, 0)
    for buf in range(NBUF):
        c = (NGROUP - 1) * NBUF + buf
        g_copy(c, buf).wait()
        p_copy(c, buf).start()
    for buf in range(NBUF):
        c = (NGROUP - 1) * NBUF + buf
        p_copy(c, buf).wait()


def kernel(input_ids, table):
    return _gather(input_ids.astype(jnp.int32), table)


# speedup vs baseline: 1.1025x; 1.1025x over previous
import functools

import jax
import jax.numpy as jnp
from jax import lax
from jax.experimental import pallas as pl
from jax.experimental.pallas import tpu as pltpu
from jax.experimental.pallas import tpu_sc as plsc

VOCAB = 30000
HIDDEN = 768
BATCH = 4
SEQ = 4096
TOTAL = BATCH * SEQ

_info = plsc.get_sparse_core_info()
NC, NS = _info.num_cores, _info.num_subcores
NW = NC * NS
ROWS_PER_W = TOTAL // NW
W_PER_B = SEQ // ROWS_PER_W
CHUNK = 32
NCHUNK = ROWS_PER_W // CHUNK
NBUF = 4
NGROUP = NCHUNK // NBUF

_mesh = plsc.VectorSubcoreMesh(core_axis_name="c", subcore_axis_name="s")


@functools.partial(
    pl.kernel,
    mesh=_mesh,
    out_type=jax.ShapeDtypeStruct((BATCH, SEQ, HIDDEN), jnp.float32),
    scratch_types=[
        pltpu.VMEM((ROWS_PER_W,), jnp.int32),
        pltpu.VMEM((NBUF, CHUNK, HIDDEN), jnp.float32),
        pltpu.SemaphoreType.DMA((NBUF,)),
        pltpu.SemaphoreType.DMA((NBUF,)),
    ],
)
def _gather(ids_hbm, table_hbm, out_hbm, idx_v, rows_v, gsem, psem):
    wid = lax.axis_index("s") * NC + lax.axis_index("c")
    b = wid // W_PER_B
    s_off = (wid % W_PER_B) * ROWS_PER_W
    pltpu.sync_copy(ids_hbm.at[b, pl.ds(s_off, ROWS_PER_W)], idx_v)

    def g_copy(c, buf):
        return pltpu.make_async_copy(
            table_hbm.at[idx_v.at[pl.ds(c * CHUNK, CHUNK)]], rows_v.at[buf],
            gsem.at[buf])

    def p_copy(c, buf):
        return pltpu.make_async_copy(
            rows_v.at[buf], out_hbm.at[b, pl.ds(s_off + c * CHUNK, CHUNK)],
            psem.at[buf])

    for buf in range(NBUF):
        g_copy(buf, buf).start()

    def group_body(g, carry):
        for buf in range(NBUF):
            c = g * NBUF + buf
            g_copy(c, buf).wait()
            p_copy(c, buf).start()
            p_copy(c, buf).wait()
            g_copy(c + NBUF, buf).start()
        return carry

    lax.fori_loop(0, NGROUP - 1, group_body, 0)
    for buf in range(NBUF):
        c = (NGROUP - 1) * NBUF + buf
        g_copy(c, buf).wait()
        p_copy(c, buf).start()
        p_copy(c, buf).wait()


def kernel(input_ids, table):
    return _gather(input_ids.astype(jnp.int32), table)
